# baseline (speedup 1.0000x reference)
import jax
import jax.numpy as jnp
from jax.experimental import pallas as pl


_G = 16


def _phi_body(v_ref, s_ref, t_ref, o_ref):
    th = t_ref[0, 0]
    c = 0.5 * th
    v = v_ref[...]
    ts_col = th * jnp.transpose(s_ref[...])
    for g in range(_G):
        for t in range(16):
            r = 32 * g + 2 * t
            x = v[r:r + 1, :]
            y = v[r + 1:r + 2, :]
            ta = c * (y - x)
            tb = c * (y + x)
            w = ta - jnp.abs(ts_col - tb)
            o_ref[g, :, 128 * t:128 * (t + 1)] = jax.nn.sigmoid(w)


def kernel(diagrams, samples, theta):
    n, p, _ = diagrams.shape
    s = samples.shape[0]

    v = diagrams.reshape(n, p // 128, 128, 2).transpose(0, 1, 3, 2)
    v = v.reshape(n * (p // 128) * 2, 128)
    s2 = samples.reshape(1, s)
    t2 = jnp.reshape(theta, (1, 1))

    out3 = pl.pallas_call(
        _phi_body,
        grid=(n // _G,),
        in_specs=[
            pl.BlockSpec(((p // 128) * 2 * _G, 128), lambda i: (i, 0)),
            pl.BlockSpec((1, s), lambda i: (0, 0)),
            pl.BlockSpec((1, 1), lambda i: (0, 0)),
        ],
        out_specs=pl.BlockSpec((_G, s, p), lambda i: (i, 0, 0)),
        out_shape=jax.ShapeDtypeStruct((n, s, p), jnp.float32),
    )(v, s2, t2)

    output = out3.transpose(0, 2, 1)
    output_shape = jnp.array(samples.shape, dtype=jnp.int32)
    return (output, output_shape)


# speedup vs baseline: 1.1752x; 1.1752x over previous
import jax
import jax.numpy as jnp
from jax.experimental import pallas as pl


_G = 8


def _phi_body(v_ref, s_ref, t_ref, o_ref):
    th = t_ref[0, 0]
    c = 0.25 * th
    v = v_ref[...]
    ts_col = (0.5 * th) * jnp.transpose(s_ref[...])
    for g in range(_G):
        for t in range(16):
            r = 32 * g + 2 * t
            x = v[r:r + 1, :]
            y = v[r + 1:r + 2, :]
            ta = c * (y - x)
            tb = c * (y + x)
            w = ta - jnp.abs(ts_col - tb)
            o_ref[g, :, 128 * t:128 * (t + 1)] = 0.5 + 0.5 * jnp.tanh(w)


def kernel(diagrams, samples, theta):
    n, p, _ = diagrams.shape
    s = samples.shape[0]

    v = diagrams.reshape(n, p // 128, 128, 2).transpose(0, 1, 3, 2)
    v = v.reshape(n * (p // 128) * 2, 128)
    s2 = samples.reshape(1, s)
    t2 = jnp.reshape(theta, (1, 1))

    out3 = pl.pallas_call(
        _phi_body,
        grid=(n // _G,),
        in_specs=[
            pl.BlockSpec(((p // 128) * 2 * _G, 128), lambda i: (i, 0)),
            pl.BlockSpec((1, s), lambda i: (0, 0)),
            pl.BlockSpec((1, 1), lambda i: (0, 0)),
        ],
        out_specs=pl.BlockSpec((_G, s, p), lambda i: (i, 0, 0)),
        out_shape=jax.ShapeDtypeStruct((n, s, p), jnp.float32),
    )(v, s2, t2)

    output = out3.transpose(0, 2, 1)
    output_shape = jnp.array(samples.shape, dtype=jnp.int32)
    return (output, output_shape)


# speedup vs baseline: 1.3921x; 1.1845x over previous
import jax
import jax.numpy as jnp
from jax.experimental import pallas as pl


_G = 8


def _phi_body(v_ref, s_ref, t_ref, o_ref, os_ref):
    th = t_ref[0, 0]
    c = 0.25 * th
    v = v_ref[...]
    ts_col = (0.5 * th) * jnp.transpose(s_ref[...])
    os_ref[...] = jnp.full((1, 1), s_ref.shape[1], jnp.int32)
    for g in range(_G):
        for t in range(16):
            r = 32 * g + 2 * t
            x = v[r:r + 1, :]
            y = v[r + 1:r + 2, :]
            ta = c * (y - x)
            tb = c * (y + x)
            w = ta - jnp.abs(ts_col - tb)
            o_ref[g, :, 128 * t:128 * (t + 1)] = 0.5 + 0.5 * jnp.tanh(w)


def kernel(diagrams, samples, theta):
    n, p, _ = diagrams.shape
    s = samples.shape[0]

    v = diagrams.reshape(n, p // 128, 128, 2).transpose(0, 1, 3, 2)
    v = v.reshape(n * (p // 128) * 2, 128)
    s2 = samples.reshape(1, s)
    t2 = jnp.reshape(theta, (1, 1))

    out3, oshape = pl.pallas_call(
        _phi_body,
        grid=(n // _G,),
        in_specs=[
            pl.BlockSpec(((p // 128) * 2 * _G, 128), lambda i: (i, 0)),
            pl.BlockSpec((1, s), lambda i: (0, 0)),
            pl.BlockSpec((1, 1), lambda i: (0, 0)),
        ],
        out_specs=[
            pl.BlockSpec((_G, s, p), lambda i: (i, 0, 0)),
            pl.BlockSpec((1, 1), lambda i: (0, 0)),
        ],
        out_shape=[
            jax.ShapeDtypeStruct((n, s, p), jnp.float32),
            jax.ShapeDtypeStruct((1, 1), jnp.int32),
        ],
    )(v, s2, t2)

    output = out3.transpose(0, 2, 1)
    output_shape = oshape.reshape(1)
    return (output, output_shape)
